# baseline (speedup 1.0000x reference)
import functools

import jax
import jax.numpy as jnp
from jax import lax
from jax.experimental import pallas as pl
from jax.experimental.pallas import tpu as pltpu
from jax.experimental.pallas import tpu_sc as plsc

_NW = 32
_L = 16
_C = 2000


@functools.partial(jax.jit, static_argnames=())
def _sc_partials(pred, label, target):
    n = pred.shape[0]
    nchunks = n // _C
    assert nchunks * _C == n

    mesh = plsc.VectorSubcoreMesh(core_axis_name="c", subcore_axis_name="s")

    @functools.partial(
        pl.kernel,
        mesh=mesh,
        out_type=[
            jax.ShapeDtypeStruct((_NW * _L,), jnp.float32),
            jax.ShapeDtypeStruct((_NW * _L,), jnp.float32),
        ],
        scratch_types=[
            pltpu.VMEM((_C,), jnp.float32),
            pltpu.VMEM((_C,), jnp.int32),
            pltpu.VMEM((_C,), jnp.float32),
            pltpu.VMEM((_L,), jnp.float32),
            pltpu.VMEM((_L,), jnp.float32),
        ],
    )
    def body(pred_h, lab_h, targ_h, loss_o, cnt_o,
             pred_v, lab_v, targ_v, stage_l, stage_c):
        wid = lax.axis_index("c") * 16 + lax.axis_index("s")
        nch = (nchunks - wid + _NW - 1) // _NW

        def vec_body(j, carry):
            acc, cv = carry
            off = j * _L
            p = pred_v[pl.ds(off, _L)]
            t = targ_v[pl.ds(off, _L)]
            lb = lab_v[pl.ds(off, _L)]
            d = p - t
            ad = jnp.abs(d)
            elem = jnp.where(ad < 1.0, 0.5 * d * d, ad - 0.5)
            m = jnp.abs(lb) == 1
            acc = acc + jnp.where(m, elem, 0.0)
            cv = cv + jnp.where(m, 1.0, 0.0)
            return acc, cv

        def chunk_body(i, carry):
            base = (wid + i * _NW) * _C
            pltpu.sync_copy(pred_h.at[pl.ds(base, _C)], pred_v)
            pltpu.sync_copy(lab_h.at[pl.ds(base, _C)], lab_v)
            pltpu.sync_copy(targ_h.at[pl.ds(base, _C)], targ_v)
            return lax.fori_loop(0, _C // _L, vec_body, carry)

        z = jnp.zeros((_L,), jnp.float32)
        acc, cv = lax.fori_loop(0, nch, chunk_body, (z, z))
        stage_l[...] = acc
        stage_c[...] = cv
        pltpu.sync_copy(stage_l, loss_o.at[pl.ds(wid * _L, _L)])
        pltpu.sync_copy(stage_c, cnt_o.at[pl.ds(wid * _L, _L)])

    return body(pred, label, target)


def kernel(pred, label, target):
    p = jnp.reshape(pred, (-1,))
    lb = label.astype(jnp.int32)
    loss_parts, cnt_parts = _sc_partials(p, lb, target)
    loss_sum = jnp.sum(loss_parts)
    cnt = jnp.sum(cnt_parts)
    return jnp.where(cnt > 0, loss_sum / jnp.maximum(cnt, 1.0),
                     jnp.float32(0.0))


# speedup vs baseline: 1.3429x; 1.3429x over previous
import functools

import jax
import jax.numpy as jnp
from jax import lax
from jax.experimental import pallas as pl
from jax.experimental.pallas import tpu as pltpu
from jax.experimental.pallas import tpu_sc as plsc

_NW = 32
_L = 16
_CHUNK = 4464
_NCHUNK = 7
_PER_W = _CHUNK * _NCHUNK


@jax.jit
def _sc_partials(pred, label, target):
    n = pred.shape[0]
    ntail = n - _NW * _PER_W
    assert 0 <= ntail // _L <= _NW and ntail % _L == 0

    mesh = plsc.VectorSubcoreMesh(core_axis_name="c", subcore_axis_name="s")

    @functools.partial(
        pl.kernel,
        mesh=mesh,
        out_type=[
            jax.ShapeDtypeStruct((_NW * _L,), jnp.float32),
            jax.ShapeDtypeStruct((_NW * _L,), jnp.float32),
        ],
        scratch_types=[
            pltpu.VMEM((_CHUNK,), jnp.float32),
            pltpu.VMEM((_CHUNK,), jnp.int32),
            pltpu.VMEM((_CHUNK,), jnp.float32),
            pltpu.VMEM((_CHUNK,), jnp.float32),
            pltpu.VMEM((_CHUNK,), jnp.int32),
            pltpu.VMEM((_CHUNK,), jnp.float32),
            pltpu.VMEM((_L,), jnp.float32),
            pltpu.VMEM((_L,), jnp.float32),
            pltpu.SemaphoreType.DMA,
            pltpu.SemaphoreType.DMA,
        ],
    )
    def body(pred_h, lab_h, targ_h, loss_o, cnt_o,
             p0, l0, t0, p1, l1, t1, stage_l, stage_c, sem0, sem1):
        wid = lax.axis_index("c") * 16 + lax.axis_index("s")
        start = wid * _PER_W
        bufs = ((p0, l0, t0, sem0), (p1, l1, t1, sem1))

        def fire(k, b):
            base = start + k * _CHUNK
            pltpu.async_copy(pred_h.at[pl.ds(base, _CHUNK)], b[0], b[3])
            pltpu.async_copy(lab_h.at[pl.ds(base, _CHUNK)], b[1], b[3])
            pltpu.async_copy(targ_h.at[pl.ds(base, _CHUNK)], b[2], b[3])

        def drain(b):
            pltpu.make_async_copy(pred_h.at[pl.ds(0, _CHUNK)], b[0], b[3]).wait()
            pltpu.make_async_copy(lab_h.at[pl.ds(0, _CHUNK)], b[1], b[3]).wait()
            pltpu.make_async_copy(targ_h.at[pl.ds(0, _CHUNK)], b[2], b[3]).wait()

        def masked_terms(p, t, lb):
            d = p - t
            ad = jnp.abs(d)
            elem = jnp.where(ad < 1.0, 0.5 * d * d, ad - 0.5)
            m = jnp.abs(lb) == 1
            return jnp.where(m, elem, 0.0), jnp.where(m, 1.0, 0.0)

        z = jnp.zeros((_L,), jnp.float32)
        acc, cv = z, z
        fire(0, bufs[0])
        for k in range(_NCHUNK):
            b = bufs[k & 1]
            if k + 1 < _NCHUNK:
                fire(k + 1, bufs[(k + 1) & 1])
            drain(b)

            def chunk_body(off, carry, b=b):
                a, c = carry
                dl, dc = masked_terms(b[0][pl.ds(off, _L)],
                                      b[2][pl.ds(off, _L)],
                                      b[1][pl.ds(off, _L)])
                return a + dl, c + dc

            acc, cv = plsc.parallel_loop(
                0, _CHUNK, _L, unroll=8, carry=(acc, cv))(chunk_body)

        stage_l[...] = acc
        stage_c[...] = cv

        @pl.when(wid < ntail // _L)
        def _():
            base = _NW * _PER_W + wid * _L
            pltpu.sync_copy(pred_h.at[pl.ds(base, _L)], p0.at[pl.ds(0, _L)])
            pltpu.sync_copy(lab_h.at[pl.ds(base, _L)], l0.at[pl.ds(0, _L)])
            pltpu.sync_copy(targ_h.at[pl.ds(base, _L)], t0.at[pl.ds(0, _L)])
            dl, dc = masked_terms(p0[pl.ds(0, _L)], t0[pl.ds(0, _L)],
                                  l0[pl.ds(0, _L)])
            stage_l[...] = stage_l[...] + dl
            stage_c[...] = stage_c[...] + dc

        pltpu.sync_copy(stage_l, loss_o.at[pl.ds(wid * _L, _L)])
        pltpu.sync_copy(stage_c, cnt_o.at[pl.ds(wid * _L, _L)])

    return body(pred, label, target)


def kernel(pred, label, target):
    p = jnp.reshape(pred, (-1,))
    lb = label.astype(jnp.int32)
    loss_parts, cnt_parts = _sc_partials(p, lb, target)
    loss_sum = jnp.sum(loss_parts)
    cnt = jnp.sum(cnt_parts)
    return jnp.where(cnt > 0, loss_sum / jnp.maximum(cnt, 1.0),
                     jnp.float32(0.0))


# speedup vs baseline: 1.4790x; 1.1014x over previous
import functools

import jax
import jax.numpy as jnp
from jax import lax
from jax.experimental import pallas as pl
from jax.experimental.pallas import tpu as pltpu
from jax.experimental.pallas import tpu_sc as plsc

_NW = 16
_L = 16
_CHUNK = 10416
_NCHUNK = 6
_PER_W = _CHUNK * _NCHUNK
_MAIN = (_CHUNK // (4 * _L)) * (4 * _L)


@jax.jit
def _sc_loss(pred, label, target):
    n = pred.shape[-1]
    ntail = n - _NW * _PER_W
    assert 0 <= ntail // _L <= _NW and ntail % _L == 0

    mesh = plsc.VectorSubcoreMesh(core_axis_name="c", subcore_axis_name="s",
                                  num_cores=1)

    @functools.partial(
        pl.kernel,
        mesh=mesh,
        compiler_params=pltpu.CompilerParams(
            needs_layout_passes=False,
            use_tc_tiling_on_sc=False,
            disable_bounds_checks=True,
            disable_semaphore_checks=True,
            skip_device_barrier=True,
        ),
        out_type=jax.ShapeDtypeStruct((2 * _L,), jnp.float32),
        scratch_types=[
            pltpu.VMEM((_CHUNK,), jnp.float32),
            pltpu.VMEM((_CHUNK,), jnp.int32),
            pltpu.VMEM((_CHUNK,), jnp.float32),
            pltpu.VMEM((_CHUNK,), jnp.float32),
            pltpu.VMEM((_CHUNK,), jnp.int32),
            pltpu.VMEM((_CHUNK,), jnp.float32),
            pltpu.VMEM((2 * _L,), jnp.float32),
            pltpu.VMEM((_NW * 2 * _L,), jnp.float32),
            pltpu.VMEM((_L,), jnp.float32),
            pltpu.VMEM_SHARED((_NW * 2 * _L,), jnp.float32),
            pltpu.SemaphoreType.DMA,
            pltpu.SemaphoreType.DMA,
        ],
    )
    def body(pred_h, lab_h, targ_h, out_h,
             p0, l0, t0, p1, l1, t1, stage, rows, tmpv, shared, sem0, sem1):
        wid = lax.axis_index("c") * _NW + lax.axis_index("s")
        start = wid * _PER_W
        bufs = ((p0, l0, t0, sem0), (p1, l1, t1, sem1))

        def fire(k, b):
            base = start + k * _CHUNK
            pltpu.async_copy(pred_h.at[0, 0, pl.ds(base, _CHUNK)], b[0], b[3])
            pltpu.async_copy(lab_h.at[pl.ds(base, _CHUNK)], b[1], b[3])
            pltpu.async_copy(targ_h.at[pl.ds(base, _CHUNK)], b[2], b[3])

        def drain(b):
            pltpu.make_async_copy(pred_h.at[0, 0, pl.ds(0, _CHUNK)], b[0], b[3]).wait()
            pltpu.make_async_copy(lab_h.at[pl.ds(0, _CHUNK)], b[1], b[3]).wait()
            pltpu.make_async_copy(targ_h.at[pl.ds(0, _CHUNK)], b[2], b[3]).wait()

        def masked_terms(p, t, lb):
            d = p - t
            ad = jnp.abs(d)
            elem = jnp.where(ad < 1.0, 0.5 * d * d, ad - 0.5)
            m = jnp.abs(lb) == 1
            return jnp.where(m, elem, 0.0), jnp.where(m, 1.0, 0.0)

        z = jnp.zeros((_L,), jnp.float32)
        accs = [z, z, z, z]
        cvs = [z, z, z, z]
        fire(0, bufs[0])
        for k in range(_NCHUNK):
            b = bufs[k & 1]
            if k + 1 < _NCHUNK:
                fire(k + 1, bufs[(k + 1) & 1])
            drain(b)

            def chunk_body(off, carry, b=b):
                a4 = list(carry[0])
                c4 = list(carry[1])
                for u in range(4):
                    dl, dc = masked_terms(b[0][pl.ds(off + u * _L, _L)],
                                          b[2][pl.ds(off + u * _L, _L)],
                                          b[1][pl.ds(off + u * _L, _L)])
                    a4[u] = a4[u] + dl
                    c4[u] = c4[u] + dc
                return tuple(a4), tuple(c4)

            accs, cvs = plsc.parallel_loop(
                0, _MAIN, 4 * _L, unroll=2,
                carry=(tuple(accs), tuple(cvs)))(chunk_body)
            accs = list(accs)
            cvs = list(cvs)
            for r in range(_MAIN, _CHUNK, _L):
                dl, dc = masked_terms(b[0][pl.ds(r, _L)],
                                      b[2][pl.ds(r, _L)],
                                      b[1][pl.ds(r, _L)])
                accs[0] = accs[0] + dl
                cvs[0] = cvs[0] + dc

        acc = (accs[0] + accs[1]) + (accs[2] + accs[3])
        cv = (cvs[0] + cvs[1]) + (cvs[2] + cvs[3])
        stage[pl.ds(0, _L)] = acc
        stage[pl.ds(_L, _L)] = cv

        @pl.when(wid < ntail // _L)
        def _():
            base = _NW * _PER_W + wid * _L
            pltpu.sync_copy(pred_h.at[0, 0, pl.ds(base, _L)], p0.at[pl.ds(0, _L)])
            pltpu.sync_copy(lab_h.at[pl.ds(base, _L)], l0.at[pl.ds(0, _L)])
            pltpu.sync_copy(targ_h.at[pl.ds(base, _L)], t0.at[pl.ds(0, _L)])
            dl, dc = masked_terms(p0[pl.ds(0, _L)], t0[pl.ds(0, _L)],
                                  l0[pl.ds(0, _L)])
            stage[pl.ds(0, _L)] = stage[pl.ds(0, _L)] + dl
            stage[pl.ds(_L, _L)] = stage[pl.ds(_L, _L)] + dc

        pltpu.sync_copy(stage, shared.at[pl.ds(wid * 2 * _L, 2 * _L)])
        plsc.subcore_barrier()

        @pl.when(wid == 0)
        def _():
            pltpu.sync_copy(shared, rows)
            a = rows[pl.ds(0, _L)]
            c = rows[pl.ds(_L, _L)]
            for r in range(1, _NW):
                a = a + rows[pl.ds(r * 2 * _L, _L)]
                c = c + rows[pl.ds(r * 2 * _L + _L, _L)]
            def allsum(v):
                for sh in (8, 4, 2, 1):
                    tmpv[...] = v
                    idx = lax.iota(jnp.int32, _L) ^ sh
                    v = v + plsc.load_gather(tmpv, [idx])
                return v

            ls = allsum(a)
            cs = allsum(c)
            res = jnp.where(cs > 0, ls / jnp.maximum(cs, 1.0),
                            jnp.zeros((_L,), jnp.float32))
            stage[pl.ds(0, _L)] = res
            stage[pl.ds(_L, _L)] = res
            pltpu.sync_copy(stage, out_h)

    return body(pred, label, target)


def kernel(pred, label, target):
    lb = label.astype(jnp.int32)
    pt = jnp.reshape(pred, (1, 1, pred.shape[0]))
    out = _sc_loss(pt, lb, target)
    return out[0]
